# baseline (speedup 1.0000x reference)
import jax
import jax.numpy as jnp
from jax.experimental import pallas as pl
from jax.experimental.pallas import tpu as pltpu

_ROWS = 3072


def _copy_kernel(in_ref, out_ref):
    out_ref[...] = in_ref[...]


def kernel(x):
    n = x.size // 1024
    xf = x.reshape(n, 1024)
    out = pl.pallas_call(
        _copy_kernel,
        grid=(n // _ROWS,),
        in_specs=[pl.BlockSpec((_ROWS, 1024), lambda i: (i, 0))],
        out_specs=pl.BlockSpec((_ROWS, 1024), lambda i: (i, 0)),
        out_shape=jax.ShapeDtypeStruct((n, 1024), x.dtype),
        compiler_params=pltpu.CompilerParams(
            dimension_semantics=("parallel",),
        ),
    )(xf)
    return out.reshape(x.shape)


# speedup vs baseline: 1.0948x; 1.0948x over previous
import functools

import jax
import jax.numpy as jnp
from jax import lax
from jax.experimental import pallas as pl
from jax.experimental.pallas import tpu as pltpu
from jax.experimental.pallas import tpu_sc as plsc

_NC = 2
_NS = 16
_NW = _NC * _NS
_CHUNK = 32768
_NBUF = 3


def _sc_copy_body(in_hbm, out_hbm, buf0, buf1, buf2, in_sems, out_sems):
    bufs = (buf0, buf1, buf2)
    n = in_hbm.shape[0]
    per = n // _NW
    niter = per // _CHUNK
    groups = niter // _NBUF
    wid = lax.axis_index("s") * _NC + lax.axis_index("c")
    base = wid * per

    def in_copy(i, b):
        return pltpu.make_async_copy(
            in_hbm.at[pl.ds(base + i * _CHUNK, _CHUNK)], bufs[b], in_sems.at[b]
        )

    def out_copy(i, b):
        return pltpu.make_async_copy(
            bufs[b], out_hbm.at[pl.ds(base + i * _CHUNK, _CHUNK)], out_sems.at[b]
        )

    for b in range(_NBUF):
        in_copy(b, b).start()

    def group(g, carry):
        for b in range(_NBUF):
            i = g * _NBUF + b
            in_copy(i, b).wait()
            out_copy(i, b).start()
            out_copy(i, b).wait()
            in_copy(i + _NBUF, b).start()
        return carry

    lax.fori_loop(0, groups - 1, group, 0)

    for b in range(_NBUF):
        i = (groups - 1) * _NBUF + b
        in_copy(i, b).wait()
        out_copy(i, b).start()
    for b in range(_NBUF):
        i = (groups - 1) * _NBUF + b
        out_copy(i, b).wait()


def _make_sc_copy(n):
    mesh = plsc.VectorSubcoreMesh(core_axis_name="c", subcore_axis_name="s")
    return pl.kernel(
        _sc_copy_body,
        out_type=jax.ShapeDtypeStruct((n,), jnp.float32),
        mesh=mesh,
        scratch_types=[
            pltpu.VMEM((_CHUNK,), jnp.float32),
            pltpu.VMEM((_CHUNK,), jnp.float32),
            pltpu.VMEM((_CHUNK,), jnp.float32),
            pltpu.SemaphoreType.DMA((_NBUF,)),
            pltpu.SemaphoreType.DMA((_NBUF,)),
        ],
    )


def kernel(x):
    n = x.size
    out = _make_sc_copy(n)(x.reshape(n))
    return out.reshape(x.shape)


# speedup vs baseline: 1.1819x; 1.0796x over previous
import functools

import jax
import jax.numpy as jnp
from jax import lax
from jax.experimental import pallas as pl
from jax.experimental.pallas import tpu as pltpu
from jax.experimental.pallas import tpu_sc as plsc

_NC = 2
_NS = 16
_NW = _NC * _NS
_CHUNK = 32768
_NBUF = 3

_SC_UNITS = 20
_TOTAL_UNITS = 96

_TC_ROWS = 4096


def _sc_copy_body(in_hbm, out_hbm, buf0, buf1, buf2, in_sems, out_sems):
    bufs = (buf0, buf1, buf2)
    n = in_hbm.shape[0]
    per = n // _NW
    niter = per // _CHUNK
    groups = niter // _NBUF
    wid = lax.axis_index("s") * _NC + lax.axis_index("c")
    base = wid * per

    def in_copy(i, b):
        return pltpu.make_async_copy(
            in_hbm.at[pl.ds(base + i * _CHUNK, _CHUNK)], bufs[b], in_sems.at[b]
        )

    def out_copy(i, b):
        return pltpu.make_async_copy(
            bufs[b], out_hbm.at[pl.ds(base + i * _CHUNK, _CHUNK)], out_sems.at[b]
        )

    for b in range(_NBUF):
        in_copy(b, b).start()

    def group(g, carry):
        for b in range(_NBUF):
            i = g * _NBUF + b
            in_copy(i, b).wait()
            out_copy(i, b).start()
            out_copy(i, b).wait()
            in_copy(i + _NBUF, b).start()
        return carry

    lax.fori_loop(0, groups - 1, group, 0)

    for b in range(_NBUF):
        i = (groups - 1) * _NBUF + b
        in_copy(i, b).wait()
        out_copy(i, b).start()
    for b in range(_NBUF):
        i = (groups - 1) * _NBUF + b
        out_copy(i, b).wait()


def _make_sc_copy(n):
    mesh = plsc.VectorSubcoreMesh(core_axis_name="c", subcore_axis_name="s")
    return pl.kernel(
        _sc_copy_body,
        out_type=jax.ShapeDtypeStruct((n,), jnp.float32),
        mesh=mesh,
        scratch_types=[
            pltpu.VMEM((_CHUNK,), jnp.float32),
            pltpu.VMEM((_CHUNK,), jnp.float32),
            pltpu.VMEM((_CHUNK,), jnp.float32),
            pltpu.SemaphoreType.DMA((_NBUF,)),
            pltpu.SemaphoreType.DMA((_NBUF,)),
        ],
    )


def _tc_copy_kernel(in_ref, out_ref):
    out_ref[...] = in_ref[...]


def _tc_copy(xf):
    n = xf.shape[0]
    return pl.pallas_call(
        _tc_copy_kernel,
        grid=(n // _TC_ROWS,),
        in_specs=[pl.BlockSpec((_TC_ROWS, 512), lambda i: (i, 0))],
        out_specs=pl.BlockSpec((_TC_ROWS, 512), lambda i: (i, 0)),
        out_shape=jax.ShapeDtypeStruct((n, 512), xf.dtype),
        compiler_params=pltpu.CompilerParams(
            dimension_semantics=("parallel",),
        ),
    )(xf)


def kernel(x):
    n = x.size
    n_sc = n // _TOTAL_UNITS * _SC_UNITS
    xf = x.reshape(n // 512, 512)
    y_sc = _make_sc_copy(n_sc)(xf[: n_sc // 512].reshape(n_sc))
    y_tc = _tc_copy(xf[n_sc // 512 :])
    out = jnp.concatenate([y_sc.reshape(n_sc // 512, 512), y_tc], axis=0)
    return out.reshape(x.shape)


# speedup vs baseline: 4.2125x; 3.5640x over previous
import jax
import jax.numpy as jnp
from jax import lax
from jax.experimental import pallas as pl
from jax.experimental.pallas import tpu as pltpu

_CHUNK_ROWS = 2048
_NSTREAM = 6


def _dma_copy_kernel(in_hbm, out_hbm, buf, in_sems, out_sems):
    nrows = in_hbm.shape[0]
    nchunks = nrows // _CHUNK_ROWS
    rounds = nchunks // _NSTREAM

    def in_copy(i, s):
        return pltpu.make_async_copy(
            in_hbm.at[pl.ds(i * _CHUNK_ROWS, _CHUNK_ROWS), :],
            buf.at[s],
            in_sems.at[s],
        )

    def out_copy(i, s):
        return pltpu.make_async_copy(
            buf.at[s],
            out_hbm.at[pl.ds(i * _CHUNK_ROWS, _CHUNK_ROWS), :],
            out_sems.at[s],
        )

    for s in range(_NSTREAM):
        in_copy(s, s).start()

    def round_body(r, carry):
        for s in range(_NSTREAM):
            i = r * _NSTREAM + s
            in_copy(i, s).wait()
            out_copy(i, s).start()
            out_copy(i, s).wait()
            in_copy(i + _NSTREAM, s).start()
        return carry

    lax.fori_loop(0, rounds - 1, round_body, 0)

    for s in range(_NSTREAM):
        i = (rounds - 1) * _NSTREAM + s
        in_copy(i, s).wait()
        out_copy(i, s).start()
    for s in range(_NSTREAM):
        i = (rounds - 1) * _NSTREAM + s
        out_copy(i, s).wait()


def kernel(x):
    n = x.size // 512
    xf = x.reshape(n, 512)
    out = pl.pallas_call(
        _dma_copy_kernel,
        in_specs=[pl.BlockSpec(memory_space=pl.ANY)],
        out_specs=pl.BlockSpec(memory_space=pl.ANY),
        scratch_shapes=[
            pltpu.VMEM((_NSTREAM, _CHUNK_ROWS, 512), jnp.float32),
            pltpu.SemaphoreType.DMA((_NSTREAM,)),
            pltpu.SemaphoreType.DMA((_NSTREAM,)),
        ],
        out_shape=jax.ShapeDtypeStruct((n, 512), x.dtype),
    )(xf)
    return out.reshape(x.shape)


# speedup vs baseline: 4.3183x; 1.0251x over previous
import jax
import jax.numpy as jnp
from jax.experimental import pallas as pl
from jax.experimental.pallas import tpu as pltpu

_ROWS = 6144


def _copy_kernel(in_ref, out_ref):
    out_ref[...] = in_ref[...]


def kernel(x):
    n = x.size // 512
    xf = x.reshape(n, 512)
    out = pl.pallas_call(
        _copy_kernel,
        grid=(n // _ROWS,),
        in_specs=[pl.BlockSpec((_ROWS, 512), lambda i: (i, 0))],
        out_specs=pl.BlockSpec((_ROWS, 512), lambda i: (i, 0)),
        out_shape=jax.ShapeDtypeStruct((n, 512), x.dtype),
        compiler_params=pltpu.CompilerParams(
            dimension_semantics=("parallel",),
        ),
    )(xf)
    return out.reshape(x.shape)


# speedup vs baseline: 4.3213x; 1.0007x over previous
import jax
import jax.numpy as jnp
from jax.experimental import pallas as pl
from jax.experimental.pallas import tpu as pltpu

_ROWS = 6144


def _copy_kernel(in_ref, out_ref):
    out_ref[...] = in_ref[...]


def kernel(x):
    n = x.size // 512
    xf = x.reshape(n, 512)
    out = pl.pallas_call(
        _copy_kernel,
        grid=(n // _ROWS,),
        in_specs=[pl.BlockSpec((_ROWS, 512), lambda i: (i, 0))],
        out_specs=pl.BlockSpec((_ROWS, 512), lambda i: (i, 0)),
        out_shape=jax.ShapeDtypeStruct((n, 512), x.dtype),
    )(xf)
    return out.reshape(x.shape)
